# baseline (speedup 1.0000x reference)
import functools
import jax
import jax.numpy as jnp
from jax import lax
from jax.experimental import pallas as pl
from jax.experimental.pallas import tpu as pltpu
from jax.experimental.pallas import tpu_sc as plsc

N = 10000
E = 320000
H = 128
ED = 20
NT = 4
ET = 8

NB = 1000
EB = 8000

NPAD = 10240
NW = 32
EW = E // NW
NSLC = NPAD // 16

_SC_MESH = plsc.VectorSubcoreMesh(core_axis_name="c", subcore_axis_name="s",
                                  num_cores=2, num_subcores=16)


def _p1_body(src_hbm, dst_hbm, ee_hbm, hs_hbm, hd_hbm,
             ex_hbm, dp_hbm,
             hs_v, hd_v, src_v, dst_v, ee_v, ex_v, den_v, red_v, out_v, shr):
    c = lax.axis_index("c")
    s = lax.axis_index("s")
    w = s * 2 + c
    base = w * EW
    pltpu.sync_copy(hs_hbm, hs_v)
    pltpu.sync_copy(hd_hbm, hd_v)
    pltpu.sync_copy(src_hbm.at[pl.ds(base, EW)], src_v)
    pltpu.sync_copy(dst_hbm.at[pl.ds(base, EW)], dst_v)
    pltpu.sync_copy(ee_hbm.at[pl.ds(base, EW)], ee_v)

    zero = jnp.zeros((16,), jnp.float32)

    def zbody(i, carry):
        den_v[pl.ds(i * 16, 16)] = zero
        return carry

    lax.fori_loop(0, NPAD // 16, zbody, 0)

    @plsc.parallel_loop(0, EW // 16, unroll=4)
    def ebody(i):
        sv = src_v[pl.ds(i * 16, 16)]
        dv = dst_v[pl.ds(i * 16, 16)]
        ev = ee_v[pl.ds(i * 16, 16)]
        hsg = plsc.load_gather(hs_v, [sv])
        hdg = plsc.load_gather(hd_v, [dv])
        logit = hsg + hdg + ev
        logit = jnp.where(logit >= 0, logit, 0.2 * logit)
        exv = jnp.exp(logit)
        ex_v[pl.ds(i * 16, 16)] = exv
        plsc.addupdate_scatter(den_v, [dv], exv)
    pltpu.sync_copy(ex_v, ex_hbm.at[pl.ds(base, EW)])

    pltpu.sync_copy(den_v, shr.at[s])
    plsc.subcore_barrier()
    pltpu.sync_copy(shr.at[:, pl.ds(s * NSLC, NSLC)], red_v)

    def rbody(j, carry):
        acc = jnp.zeros((16,), jnp.float32)
        for r in range(16):
            acc = acc + red_v[r, pl.ds(j * 16, 16)]
        out_v[pl.ds(j * 16, 16)] = acc
        return carry

    lax.fori_loop(0, NSLC // 16, rbody, 0)
    pltpu.sync_copy(out_v, dp_hbm.at[pl.ds(c * NPAD + s * NSLC, NSLC)])


def _pass1(src, dst, ee, hs_pad, hd_pad):
    return pl.kernel(
        _p1_body,
        out_type=[
            jax.ShapeDtypeStruct((E,), jnp.float32),
            jax.ShapeDtypeStruct((2 * NPAD,), jnp.float32),
        ],
        mesh=_SC_MESH,
        scratch_types=[
            pltpu.VMEM((NPAD,), jnp.float32),
            pltpu.VMEM((NPAD,), jnp.float32),
            pltpu.VMEM((EW,), jnp.int32),
            pltpu.VMEM((EW,), jnp.int32),
            pltpu.VMEM((EW,), jnp.float32),
            pltpu.VMEM((EW,), jnp.float32),
            pltpu.VMEM((NPAD,), jnp.float32),
            pltpu.VMEM((16, NSLC), jnp.float32),
            pltpu.VMEM((NSLC,), jnp.float32),
            pltpu.VMEM_SHARED((16, NPAD), jnp.float32),
        ],
        compiler_params=pltpu.CompilerParams(needs_layout_passes=False),
    )(src, dst, ee, hs_pad, hd_pad)


CH2 = 4000
NCH2 = E // CH2
CH3 = 2000
NCH3 = E // CH3
FC = H // NW


def _recip_denom(stage_v, den_v):
    def body(j, carry):
        d = stage_v[pl.ds(j * 16, 16)] + stage_v[pl.ds(NPAD + j * 16, 16)]
        den_v[pl.ds(j * 16, 16)] = 1.0 / (d + 1e-16)
        return carry

    lax.fori_loop(0, NPAD // 16, body, 0)


def _p2_body(src_hbm, dst_hbm, ex_hbm, dp_hbm, hT_hbm,
             outT_hbm,
             h_v, o_v, den_v, src_c0, dst_c0, ex_c0, src_c1, dst_c1, ex_c1,
             sem0, sem1):
    c = lax.axis_index("c")
    s = lax.axis_index("s")
    w = s * 2 + c
    pltpu.sync_copy(hT_hbm.at[pl.ds(w * (NPAD * FC), NPAD * FC)], h_v)
    pltpu.sync_copy(dp_hbm, o_v.at[pl.ds(0, 2 * NPAD)])
    _recip_denom(o_v, den_v)

    zero = jnp.zeros((16,), jnp.float32)

    def zbody(i, carry):
        o_v[pl.ds(i * 16, 16)] = zero
        return carry

    lax.fori_loop(0, NPAD * FC // 16, zbody, 0)

    bufs = ((src_c0, dst_c0, ex_c0, sem0), (src_c1, dst_c1, ex_c1, sem1))

    def start(k, bi):
        sc, dc, ec, sem = bufs[bi]
        base = k * CH2
        pltpu.async_copy(src_hbm.at[pl.ds(base, CH2)], sc, sem)
        pltpu.async_copy(dst_hbm.at[pl.ds(base, CH2)], dc, sem)
        pltpu.async_copy(ex_hbm.at[pl.ds(base, CH2)], ec, sem)

    def drain(bi):
        sc, dc, ec, sem = bufs[bi]
        pltpu.make_async_copy(src_hbm.at[pl.ds(0, CH2)], sc, sem).wait()
        pltpu.make_async_copy(dst_hbm.at[pl.ds(0, CH2)], dc, sem).wait()
        pltpu.make_async_copy(ex_hbm.at[pl.ds(0, CH2)], ec, sem).wait()

    start(0, 0)

    def pair(kk, carry):
        for b in range(2):
            k = kk * 2 + b
            drain(b)

            @pl.when(k + 1 < NCH2)
            def _():
                start(k + 1, 1 - b)

            sc, dc, ec, _sem = bufs[b]

            @plsc.parallel_loop(0, CH2 // 16, unroll=8)
            def step(i):
                sv = sc[pl.ds(i * 16, 16)]
                dv = dc[pl.ds(i * 16, 16)]
                exv = ec[pl.ds(i * 16, 16)]
                av = exv * plsc.load_gather(den_v, [dv])
                for colc in range(FC):
                    g = plsc.load_gather(h_v, [sv + colc * NPAD])
                    plsc.addupdate_scatter(o_v, [dv + colc * NPAD], av * g)
        return carry

    lax.fori_loop(0, NCH2 // 2, pair, 0)
    pltpu.sync_copy(o_v, outT_hbm.at[pl.ds(w * (NPAD * FC), NPAD * FC)])


def _pass2(src, dst, ex, dp, hT):
    return pl.kernel(
        _p2_body,
        out_type=jax.ShapeDtypeStruct((NW * NPAD * FC,), jnp.float32),
        mesh=_SC_MESH,
        scratch_types=[
            pltpu.VMEM((NPAD * FC,), jnp.float32),
            pltpu.VMEM((NPAD * FC,), jnp.float32),
            pltpu.VMEM((NPAD,), jnp.float32),
            pltpu.VMEM((CH2,), jnp.int32),
            pltpu.VMEM((CH2,), jnp.int32),
            pltpu.VMEM((CH2,), jnp.float32),
            pltpu.VMEM((CH2,), jnp.int32),
            pltpu.VMEM((CH2,), jnp.int32),
            pltpu.VMEM((CH2,), jnp.float32),
            pltpu.SemaphoreType.DMA,
            pltpu.SemaphoreType.DMA,
        ],
        compiler_params=pltpu.CompilerParams(needs_layout_passes=False),
    )(src, dst, ex, dp, hT)


def _p3_body(dst_hbm, et_hbm, ex_hbm, dp_hbm, eaT_hbm,
             zT_hbm,
             z_v, den_v, dst_c0, et_c0, ex_c0, ea_c0,
             dst_c1, et_c1, ex_c1, ea_c1, sem0, sem1):
    c = lax.axis_index("c")
    s = lax.axis_index("s")
    w = s * 2 + c

    @pl.when(w < ED)
    def _():
        pltpu.sync_copy(dp_hbm, z_v.at[pl.ds(0, 2 * NPAD)])
        _recip_denom(z_v, den_v)

        zero = jnp.zeros((16,), jnp.float32)

        def zbody(i, carry):
            z_v[pl.ds(i * 16, 16)] = zero
            return carry

        lax.fori_loop(0, ET * NPAD // 16, zbody, 0)

        bufs = ((dst_c0, et_c0, ex_c0, ea_c0, sem0),
                (dst_c1, et_c1, ex_c1, ea_c1, sem1))

        def start(k, bi):
            dc, tc, ec, ac, sem = bufs[bi]
            base = k * CH3
            pltpu.async_copy(dst_hbm.at[pl.ds(base, CH3)], dc, sem)
            pltpu.async_copy(et_hbm.at[pl.ds(base, CH3)], tc, sem)
            pltpu.async_copy(ex_hbm.at[pl.ds(base, CH3)], ec, sem)
            pltpu.async_copy(eaT_hbm.at[pl.ds(w * E + base, CH3)], ac, sem)

        def drain(bi):
            dc, tc, ec, ac, sem = bufs[bi]
            pltpu.make_async_copy(dst_hbm.at[pl.ds(0, CH3)], dc, sem).wait()
            pltpu.make_async_copy(et_hbm.at[pl.ds(0, CH3)], tc, sem).wait()
            pltpu.make_async_copy(ex_hbm.at[pl.ds(0, CH3)], ec, sem).wait()
            pltpu.make_async_copy(eaT_hbm.at[pl.ds(0, CH3)], ac, sem).wait()

        start(0, 0)

        def pair(kk, carry):
            for b in range(2):
                k = kk * 2 + b
                drain(b)

                @pl.when(k + 1 < NCH3)
                def _():
                    start(k + 1, 1 - b)

                dc, tc, ec, ac, _sem = bufs[b]

                @plsc.parallel_loop(0, CH3 // 16, unroll=8)
                def step(i):
                    dv = dc[pl.ds(i * 16, 16)]
                    tv = tc[pl.ds(i * 16, 16)]
                    exv = ec[pl.ds(i * 16, 16)]
                    eav = ac[pl.ds(i * 16, 16)]
                    av = exv * plsc.load_gather(den_v, [dv])
                    plsc.addupdate_scatter(z_v, [tv * NPAD + dv], av * eav)
            return carry

        lax.fori_loop(0, NCH3 // 2, pair, 0)
        pltpu.sync_copy(z_v, zT_hbm.at[pl.ds(w * (ET * NPAD), ET * NPAD)])


def _pass3(dst, edge_type, ex, dp, eaT):
    return pl.kernel(
        _p3_body,
        out_type=jax.ShapeDtypeStruct((ED * ET * NPAD,), jnp.float32),
        mesh=_SC_MESH,
        scratch_types=[
            pltpu.VMEM((ET * NPAD,), jnp.float32),
            pltpu.VMEM((NPAD,), jnp.float32),
            pltpu.VMEM((CH3,), jnp.int32),
            pltpu.VMEM((CH3,), jnp.int32),
            pltpu.VMEM((CH3,), jnp.float32),
            pltpu.VMEM((CH3,), jnp.float32),
            pltpu.VMEM((CH3,), jnp.int32),
            pltpu.VMEM((CH3,), jnp.int32),
            pltpu.VMEM((CH3,), jnp.float32),
            pltpu.VMEM((CH3,), jnp.float32),
            pltpu.SemaphoreType.DMA,
            pltpu.SemaphoreType.DMA,
        ],
        compiler_params=pltpu.CompilerParams(needs_layout_passes=False),
    )(dst, edge_type, ex, dp, eaT)


def _node_prep_body(x_ref, nt_ref, Wn_ref, asrc_ref, adst_ref,
                    h_ref, hs_ref, hd_ref):
    xb = x_ref[...]
    nt = nt_ref[...]
    acc = jnp.zeros((NB, H), jnp.float32)
    for t in range(NT):
        acc = acc + jnp.where(nt == t, xb @ Wn_ref[t], 0.0)
    h_ref[...] = acc
    hs_ref[...] = acc @ asrc_ref[...]
    hd_ref[...] = acc @ adst_ref[...]


def _node_prep(x, node_type2d, W_node, a_src2d, a_dst2d):
    grid = (N // NB,)
    return pl.pallas_call(
        _node_prep_body,
        grid=grid,
        in_specs=[
            pl.BlockSpec((NB, H), lambda i: (i, 0)),
            pl.BlockSpec((NB, 1), lambda i: (i, 0)),
            pl.BlockSpec((NT, H, H), lambda i: (0, 0, 0)),
            pl.BlockSpec((H, 1), lambda i: (0, 0)),
            pl.BlockSpec((H, 1), lambda i: (0, 0)),
        ],
        out_specs=[
            pl.BlockSpec((NB, H), lambda i: (i, 0)),
            pl.BlockSpec((NB, 1), lambda i: (i, 0)),
            pl.BlockSpec((NB, 1), lambda i: (i, 0)),
        ],
        out_shape=[
            jax.ShapeDtypeStruct((N, H), jnp.float32),
            jax.ShapeDtypeStruct((N, 1), jnp.float32),
            jax.ShapeDtypeStruct((N, 1), jnp.float32),
        ],
    )(x, node_type2d, W_node, a_src2d, a_dst2d)


def _edge_prep_body(ea_ref, et_ref, We1_ref, ae1_ref, We2_ref, ae2_ref,
                    ee1_ref, ee2_ref):
    ea = ea_ref[...]
    et = et_ref[...]
    oh = (et == lax.broadcasted_iota(jnp.int32, (1, ET), 1)).astype(jnp.float32)
    v1 = (We1_ref[...].reshape(ET * ED, H) @ ae1_ref[...]).reshape(ET, ED)
    v2 = (We2_ref[...].reshape(ET * ED, H) @ ae2_ref[...]).reshape(ET, ED)
    vg1 = oh @ v1
    vg2 = oh @ v2
    ee1_ref[...] = jnp.sum(ea * vg1, axis=1, keepdims=True)
    ee2_ref[...] = jnp.sum(ea * vg2, axis=1, keepdims=True)


def _edge_prep(edge_attr, edge_type2d, We1, ae1, We2, ae2):
    grid = (E // EB,)
    return pl.pallas_call(
        _edge_prep_body,
        grid=grid,
        in_specs=[
            pl.BlockSpec((EB, ED), lambda i: (i, 0)),
            pl.BlockSpec((EB, 1), lambda i: (i, 0)),
            pl.BlockSpec((ET, ED, H), lambda i: (0, 0, 0)),
            pl.BlockSpec((H, 1), lambda i: (0, 0)),
            pl.BlockSpec((ET, ED, H), lambda i: (0, 0, 0)),
            pl.BlockSpec((H, 1), lambda i: (0, 0)),
        ],
        out_specs=[
            pl.BlockSpec((EB, 1), lambda i: (i, 0)),
            pl.BlockSpec((EB, 1), lambda i: (i, 0)),
        ],
        out_shape=[
            jax.ShapeDtypeStruct((E, 1), jnp.float32),
            jax.ShapeDtypeStruct((E, 1), jnp.float32),
        ],
    )(edge_attr, edge_type2d, We1, ae1, We2, ae2)


NBC = 1024


def _combine_body(om_ref, zT_ref, W2_ref, b_ref, out_ref):
    oe = lax.dot_general(zT_ref[...], W2_ref[...], (((0,), (0,)), ((), ())),
                         preferred_element_type=jnp.float32)
    out_ref[...] = om_ref[...] + oe + b_ref[...]


def _combine(om_pad, zT2, W2, bias2d):
    grid = (NPAD // NBC,)
    return pl.pallas_call(
        _combine_body,
        grid=grid,
        in_specs=[
            pl.BlockSpec((NBC, H), lambda i: (i, 0)),
            pl.BlockSpec((ED * ET, NBC), lambda i: (0, i)),
            pl.BlockSpec((ED * ET, H), lambda i: (0, 0)),
            pl.BlockSpec((1, H), lambda i: (0, 0)),
        ],
        out_specs=pl.BlockSpec((NBC, H), lambda i: (i, 0)),
        out_shape=jax.ShapeDtypeStruct((NPAD, H), jnp.float32),
    )(om_pad, zT2, W2, bias2d)


def _layer(x, src, dst, node_type2d, edge_type, eaT, ee, p):
    h, hs, hd = _node_prep(x, node_type2d, p["W_node"],
                           p["a_src"].reshape(H, 1), p["a_dst"].reshape(H, 1))
    hs_pad = jnp.pad(hs[:, 0], (0, NPAD - N))
    hd_pad = jnp.pad(hd[:, 0], (0, NPAD - N))
    ex, dp = _pass1(src, dst, ee, hs_pad, hd_pad)
    hT = (jnp.pad(h, ((0, NPAD - N), (0, 0)))
          .reshape(NPAD, NW, FC).transpose(1, 2, 0).reshape(NW * NPAD * FC))
    outT = _pass2(src, dst, ex, dp, hT)
    om_pad = outT.reshape(NW, FC, NPAD).transpose(2, 0, 1).reshape(NPAD, H)
    zT = _pass3(dst, edge_type, ex, dp, eaT)
    zT2 = zT.reshape(ED * ET, NPAD)
    W2 = p["W_edge"].transpose(1, 0, 2).reshape(ED * ET, H)
    return _combine(om_pad, zT2, W2, p["bias"].reshape(1, H))[:N]


def kernel(x, edge_index, node_type, edge_attr, edge_type, params1, params2):
    src = edge_index[0]
    dst = edge_index[1]
    node_type2d = node_type.reshape(N, 1)
    edge_type2d = edge_type.reshape(E, 1)
    eaT = edge_attr.T.reshape(ED * E)
    ee1, ee2 = _edge_prep(edge_attr, edge_type2d,
                          params1["W_edge"], params1["a_edge"].reshape(H, 1),
                          params2["W_edge"], params2["a_edge"].reshape(H, 1))
    h1 = _layer(x, src, dst, node_type2d, edge_type, eaT, ee1[:, 0], params1)
    h2 = _layer(h1, src, dst, node_type2d, edge_type, eaT, ee2[:, 0], params2)
    return h2


# speedup vs baseline: 1.0876x; 1.0876x over previous
import functools
import jax
import jax.numpy as jnp
from jax import lax
from jax.experimental import pallas as pl
from jax.experimental.pallas import tpu as pltpu
from jax.experimental.pallas import tpu_sc as plsc

N = 10000
E = 320000
H = 128
ED = 20
NT = 4
ET = 8

NB = 1000
EB = 8000

NPAD = 10240
NW = 32
EW = E // NW
NSLC = NPAD // 16

_SC_MESH = plsc.VectorSubcoreMesh(core_axis_name="c", subcore_axis_name="s",
                                  num_cores=2, num_subcores=16)


def _p1_body(src_hbm, dst_hbm, ee_hbm, hs_hbm, hd_hbm,
             ex_hbm, dp_hbm,
             hs_v, hd_v, src_v, dst_v, ee_v, ex_v, den_v, red_v, out_v, shr):
    c = lax.axis_index("c")
    s = lax.axis_index("s")
    w = s * 2 + c
    base = w * EW
    pltpu.sync_copy(hs_hbm, hs_v)
    pltpu.sync_copy(hd_hbm, hd_v)
    pltpu.sync_copy(src_hbm.at[pl.ds(base, EW)], src_v)
    pltpu.sync_copy(dst_hbm.at[pl.ds(base, EW)], dst_v)
    pltpu.sync_copy(ee_hbm.at[pl.ds(base, EW)], ee_v)

    zero = jnp.zeros((16,), jnp.float32)

    def zbody(i, carry):
        den_v[pl.ds(i * 16, 16)] = zero
        return carry

    lax.fori_loop(0, NPAD // 16, zbody, 0)

    @plsc.parallel_loop(0, EW // 16, unroll=4)
    def ebody(i):
        sv = src_v[pl.ds(i * 16, 16)]
        dv = dst_v[pl.ds(i * 16, 16)]
        ev = ee_v[pl.ds(i * 16, 16)]
        hsg = plsc.load_gather(hs_v, [sv])
        hdg = plsc.load_gather(hd_v, [dv])
        logit = hsg + hdg + ev
        logit = jnp.where(logit >= 0, logit, 0.2 * logit)
        exv = jnp.exp(logit)
        ex_v[pl.ds(i * 16, 16)] = exv
        plsc.addupdate_scatter(den_v, [dv], exv)
    pltpu.sync_copy(ex_v, ex_hbm.at[pl.ds(base, EW)])

    pltpu.sync_copy(den_v, shr.at[s])
    plsc.subcore_barrier()
    pltpu.sync_copy(shr.at[:, pl.ds(s * NSLC, NSLC)], red_v)

    def rbody(j, carry):
        acc = jnp.zeros((16,), jnp.float32)
        for r in range(16):
            acc = acc + red_v[r, pl.ds(j * 16, 16)]
        out_v[pl.ds(j * 16, 16)] = acc
        return carry

    lax.fori_loop(0, NSLC // 16, rbody, 0)
    pltpu.sync_copy(out_v, dp_hbm.at[pl.ds(c * NPAD + s * NSLC, NSLC)])


def _pass1(src, dst, ee, hs_pad, hd_pad):
    return pl.kernel(
        _p1_body,
        out_type=[
            jax.ShapeDtypeStruct((E,), jnp.float32),
            jax.ShapeDtypeStruct((2 * NPAD,), jnp.float32),
        ],
        mesh=_SC_MESH,
        scratch_types=[
            pltpu.VMEM((NPAD,), jnp.float32),
            pltpu.VMEM((NPAD,), jnp.float32),
            pltpu.VMEM((EW,), jnp.int32),
            pltpu.VMEM((EW,), jnp.int32),
            pltpu.VMEM((EW,), jnp.float32),
            pltpu.VMEM((EW,), jnp.float32),
            pltpu.VMEM((NPAD,), jnp.float32),
            pltpu.VMEM((16, NSLC), jnp.float32),
            pltpu.VMEM((NSLC,), jnp.float32),
            pltpu.VMEM_SHARED((16, NPAD), jnp.float32),
        ],
        compiler_params=pltpu.CompilerParams(needs_layout_passes=False),
    )(src, dst, ee, hs_pad, hd_pad)


CH2 = 4000
NCH2 = E // CH2
CH3 = 2000
NCH3 = E // CH3
FC = H // NW


def _recip_denom(stage_v, den_v):
    def body(j, carry):
        d = stage_v[pl.ds(j * 16, 16)] + stage_v[pl.ds(NPAD + j * 16, 16)]
        den_v[pl.ds(j * 16, 16)] = 1.0 / (d + 1e-16)
        return carry

    lax.fori_loop(0, NPAD // 16, body, 0)


def _p2_body(src_hbm, dst_hbm, ex_hbm, dp_hbm, hT_hbm,
             outT_hbm,
             h_v, o_v, den_v, src_c0, dst_c0, ex_c0, src_c1, dst_c1, ex_c1,
             sem0, sem1):
    c = lax.axis_index("c")
    s = lax.axis_index("s")
    w = s * 2 + c
    pltpu.sync_copy(hT_hbm.at[pl.ds(w * (NPAD * FC), NPAD * FC)], h_v)
    pltpu.sync_copy(dp_hbm, o_v.at[pl.ds(0, 2 * NPAD)])
    _recip_denom(o_v, den_v)

    zero = jnp.zeros((16,), jnp.float32)

    def zbody(i, carry):
        o_v[pl.ds(i * 16, 16)] = zero
        return carry

    lax.fori_loop(0, NPAD * FC // 16, zbody, 0)

    bufs = ((src_c0, dst_c0, ex_c0, sem0), (src_c1, dst_c1, ex_c1, sem1))

    def start(k, bi):
        sc, dc, ec, sem = bufs[bi]
        base = k * CH2
        pltpu.async_copy(src_hbm.at[pl.ds(base, CH2)], sc, sem)
        pltpu.async_copy(dst_hbm.at[pl.ds(base, CH2)], dc, sem)
        pltpu.async_copy(ex_hbm.at[pl.ds(base, CH2)], ec, sem)

    def drain(bi):
        sc, dc, ec, sem = bufs[bi]
        pltpu.make_async_copy(src_hbm.at[pl.ds(0, CH2)], sc, sem).wait()
        pltpu.make_async_copy(dst_hbm.at[pl.ds(0, CH2)], dc, sem).wait()
        pltpu.make_async_copy(ex_hbm.at[pl.ds(0, CH2)], ec, sem).wait()

    start(0, 0)

    def pair(kk, carry):
        for b in range(2):
            k = kk * 2 + b
            drain(b)

            @pl.when(k + 1 < NCH2)
            def _():
                start(k + 1, 1 - b)

            sc, dc, ec, _sem = bufs[b]

            @plsc.parallel_loop(0, CH2 // 16, unroll=4)
            def step(i):
                sv = sc[pl.ds(i * 16, 16)]
                dv = dc[pl.ds(i * 16, 16)]
                exv = ec[pl.ds(i * 16, 16)]
                av = exv * plsc.load_gather(den_v, [dv])
                for colc in range(FC):
                    g = plsc.load_gather(h_v, [sv + colc * NPAD])
                    plsc.addupdate_scatter(o_v, [dv + colc * NPAD], av * g)
        return carry

    lax.fori_loop(0, NCH2 // 2, pair, 0)
    pltpu.sync_copy(o_v, outT_hbm.at[pl.ds(w * (NPAD * FC), NPAD * FC)])


def _pass2(src, dst, ex, dp, hT):
    return pl.kernel(
        _p2_body,
        out_type=jax.ShapeDtypeStruct((NW * NPAD * FC,), jnp.float32),
        mesh=_SC_MESH,
        scratch_types=[
            pltpu.VMEM((NPAD * FC,), jnp.float32),
            pltpu.VMEM((NPAD * FC,), jnp.float32),
            pltpu.VMEM((NPAD,), jnp.float32),
            pltpu.VMEM((CH2,), jnp.int32),
            pltpu.VMEM((CH2,), jnp.int32),
            pltpu.VMEM((CH2,), jnp.float32),
            pltpu.VMEM((CH2,), jnp.int32),
            pltpu.VMEM((CH2,), jnp.int32),
            pltpu.VMEM((CH2,), jnp.float32),
            pltpu.SemaphoreType.DMA,
            pltpu.SemaphoreType.DMA,
        ],
        compiler_params=pltpu.CompilerParams(needs_layout_passes=False),
    )(src, dst, ex, dp, hT)


def _p3_body(dst_hbm, et_hbm, ex_hbm, dp_hbm, eaT_hbm,
             zT_hbm,
             z_v, den_v, dst_c0, et_c0, ex_c0, ea_c0,
             dst_c1, et_c1, ex_c1, ea_c1, sem0, sem1):
    c = lax.axis_index("c")
    s = lax.axis_index("s")
    w = s * 2 + c

    @pl.when(w < ED)
    def _():
        pltpu.sync_copy(dp_hbm, z_v.at[pl.ds(0, 2 * NPAD)])
        _recip_denom(z_v, den_v)

        zero = jnp.zeros((16,), jnp.float32)

        def zbody(i, carry):
            z_v[pl.ds(i * 16, 16)] = zero
            return carry

        lax.fori_loop(0, ET * NPAD // 16, zbody, 0)

        bufs = ((dst_c0, et_c0, ex_c0, ea_c0, sem0),
                (dst_c1, et_c1, ex_c1, ea_c1, sem1))

        def start(k, bi):
            dc, tc, ec, ac, sem = bufs[bi]
            base = k * CH3
            pltpu.async_copy(dst_hbm.at[pl.ds(base, CH3)], dc, sem)
            pltpu.async_copy(et_hbm.at[pl.ds(base, CH3)], tc, sem)
            pltpu.async_copy(ex_hbm.at[pl.ds(base, CH3)], ec, sem)
            pltpu.async_copy(eaT_hbm.at[pl.ds(w * E + base, CH3)], ac, sem)

        def drain(bi):
            dc, tc, ec, ac, sem = bufs[bi]
            pltpu.make_async_copy(dst_hbm.at[pl.ds(0, CH3)], dc, sem).wait()
            pltpu.make_async_copy(et_hbm.at[pl.ds(0, CH3)], tc, sem).wait()
            pltpu.make_async_copy(ex_hbm.at[pl.ds(0, CH3)], ec, sem).wait()
            pltpu.make_async_copy(eaT_hbm.at[pl.ds(0, CH3)], ac, sem).wait()

        start(0, 0)

        def pair(kk, carry):
            for b in range(2):
                k = kk * 2 + b
                drain(b)

                @pl.when(k + 1 < NCH3)
                def _():
                    start(k + 1, 1 - b)

                dc, tc, ec, ac, _sem = bufs[b]

                @plsc.parallel_loop(0, CH3 // 16, unroll=4)
                def step(i):
                    dv = dc[pl.ds(i * 16, 16)]
                    tv = tc[pl.ds(i * 16, 16)]
                    exv = ec[pl.ds(i * 16, 16)]
                    eav = ac[pl.ds(i * 16, 16)]
                    av = exv * plsc.load_gather(den_v, [dv])
                    plsc.addupdate_scatter(z_v, [tv * NPAD + dv], av * eav)
            return carry

        lax.fori_loop(0, NCH3 // 2, pair, 0)
        pltpu.sync_copy(z_v, zT_hbm.at[pl.ds(w * (ET * NPAD), ET * NPAD)])


def _pass3(dst, edge_type, ex, dp, eaT):
    return pl.kernel(
        _p3_body,
        out_type=jax.ShapeDtypeStruct((ED * ET * NPAD,), jnp.float32),
        mesh=_SC_MESH,
        scratch_types=[
            pltpu.VMEM((ET * NPAD,), jnp.float32),
            pltpu.VMEM((NPAD,), jnp.float32),
            pltpu.VMEM((CH3,), jnp.int32),
            pltpu.VMEM((CH3,), jnp.int32),
            pltpu.VMEM((CH3,), jnp.float32),
            pltpu.VMEM((CH3,), jnp.float32),
            pltpu.VMEM((CH3,), jnp.int32),
            pltpu.VMEM((CH3,), jnp.int32),
            pltpu.VMEM((CH3,), jnp.float32),
            pltpu.VMEM((CH3,), jnp.float32),
            pltpu.SemaphoreType.DMA,
            pltpu.SemaphoreType.DMA,
        ],
        compiler_params=pltpu.CompilerParams(needs_layout_passes=False),
    )(dst, edge_type, ex, dp, eaT)


def _node_prep_body(x_ref, nt_ref, Wn_ref, asrc_ref, adst_ref,
                    h_ref, hs_ref, hd_ref):
    xb = x_ref[...]
    nt = nt_ref[...]
    acc = jnp.zeros((NB, H), jnp.float32)
    for t in range(NT):
        acc = acc + jnp.where(nt == t, xb @ Wn_ref[t], 0.0)
    h_ref[...] = acc
    hs_ref[...] = acc @ asrc_ref[...]
    hd_ref[...] = acc @ adst_ref[...]


def _node_prep(x, node_type2d, W_node, a_src2d, a_dst2d):
    grid = (N // NB,)
    return pl.pallas_call(
        _node_prep_body,
        grid=grid,
        in_specs=[
            pl.BlockSpec((NB, H), lambda i: (i, 0)),
            pl.BlockSpec((NB, 1), lambda i: (i, 0)),
            pl.BlockSpec((NT, H, H), lambda i: (0, 0, 0)),
            pl.BlockSpec((H, 1), lambda i: (0, 0)),
            pl.BlockSpec((H, 1), lambda i: (0, 0)),
        ],
        out_specs=[
            pl.BlockSpec((NB, H), lambda i: (i, 0)),
            pl.BlockSpec((NB, 1), lambda i: (i, 0)),
            pl.BlockSpec((NB, 1), lambda i: (i, 0)),
        ],
        out_shape=[
            jax.ShapeDtypeStruct((N, H), jnp.float32),
            jax.ShapeDtypeStruct((N, 1), jnp.float32),
            jax.ShapeDtypeStruct((N, 1), jnp.float32),
        ],
    )(x, node_type2d, W_node, a_src2d, a_dst2d)


def _edge_prep_body(ea_ref, et_ref, We1_ref, ae1_ref, We2_ref, ae2_ref,
                    ee1_ref, ee2_ref):
    ea = ea_ref[...]
    et = et_ref[...]
    oh = (et == lax.broadcasted_iota(jnp.int32, (1, ET), 1)).astype(jnp.float32)
    v1 = (We1_ref[...].reshape(ET * ED, H) @ ae1_ref[...]).reshape(ET, ED)
    v2 = (We2_ref[...].reshape(ET * ED, H) @ ae2_ref[...]).reshape(ET, ED)
    vg1 = oh @ v1
    vg2 = oh @ v2
    ee1_ref[...] = jnp.sum(ea * vg1, axis=1, keepdims=True)
    ee2_ref[...] = jnp.sum(ea * vg2, axis=1, keepdims=True)


def _edge_prep(edge_attr, edge_type2d, We1, ae1, We2, ae2):
    grid = (E // EB,)
    return pl.pallas_call(
        _edge_prep_body,
        grid=grid,
        in_specs=[
            pl.BlockSpec((EB, ED), lambda i: (i, 0)),
            pl.BlockSpec((EB, 1), lambda i: (i, 0)),
            pl.BlockSpec((ET, ED, H), lambda i: (0, 0, 0)),
            pl.BlockSpec((H, 1), lambda i: (0, 0)),
            pl.BlockSpec((ET, ED, H), lambda i: (0, 0, 0)),
            pl.BlockSpec((H, 1), lambda i: (0, 0)),
        ],
        out_specs=[
            pl.BlockSpec((EB, 1), lambda i: (i, 0)),
            pl.BlockSpec((EB, 1), lambda i: (i, 0)),
        ],
        out_shape=[
            jax.ShapeDtypeStruct((E, 1), jnp.float32),
            jax.ShapeDtypeStruct((E, 1), jnp.float32),
        ],
    )(edge_attr, edge_type2d, We1, ae1, We2, ae2)


NBC = 1024


def _combine_body(om_ref, zT_ref, W2_ref, b_ref, out_ref):
    oe = lax.dot_general(zT_ref[...], W2_ref[...], (((0,), (0,)), ((), ())),
                         preferred_element_type=jnp.float32)
    out_ref[...] = om_ref[...] + oe + b_ref[...]


def _combine(om_pad, zT2, W2, bias2d):
    grid = (NPAD // NBC,)
    return pl.pallas_call(
        _combine_body,
        grid=grid,
        in_specs=[
            pl.BlockSpec((NBC, H), lambda i: (i, 0)),
            pl.BlockSpec((ED * ET, NBC), lambda i: (0, i)),
            pl.BlockSpec((ED * ET, H), lambda i: (0, 0)),
            pl.BlockSpec((1, H), lambda i: (0, 0)),
        ],
        out_specs=pl.BlockSpec((NBC, H), lambda i: (i, 0)),
        out_shape=jax.ShapeDtypeStruct((NPAD, H), jnp.float32),
    )(om_pad, zT2, W2, bias2d)


def _layer(x, src, dst, node_type2d, edge_type, eaT, ee, p):
    h, hs, hd = _node_prep(x, node_type2d, p["W_node"],
                           p["a_src"].reshape(H, 1), p["a_dst"].reshape(H, 1))
    hs_pad = jnp.pad(hs[:, 0], (0, NPAD - N))
    hd_pad = jnp.pad(hd[:, 0], (0, NPAD - N))
    ex, dp = _pass1(src, dst, ee, hs_pad, hd_pad)
    hT = (jnp.pad(h, ((0, NPAD - N), (0, 0)))
          .reshape(NPAD, NW, FC).transpose(1, 2, 0).reshape(NW * NPAD * FC))
    outT = _pass2(src, dst, ex, dp, hT)
    om_pad = outT.reshape(NW, FC, NPAD).transpose(2, 0, 1).reshape(NPAD, H)
    zT = _pass3(dst, edge_type, ex, dp, eaT)
    zT2 = zT.reshape(ED * ET, NPAD)
    W2 = p["W_edge"].transpose(1, 0, 2).reshape(ED * ET, H)
    return _combine(om_pad, zT2, W2, p["bias"].reshape(1, H))[:N]


def kernel(x, edge_index, node_type, edge_attr, edge_type, params1, params2):
    src = edge_index[0]
    dst = edge_index[1]
    node_type2d = node_type.reshape(N, 1)
    edge_type2d = edge_type.reshape(E, 1)
    eaT = edge_attr.T.reshape(ED * E)
    ee1, ee2 = _edge_prep(edge_attr, edge_type2d,
                          params1["W_edge"], params1["a_edge"].reshape(H, 1),
                          params2["W_edge"], params2["a_edge"].reshape(H, 1))
    h1 = _layer(x, src, dst, node_type2d, edge_type, eaT, ee1[:, 0], params1)
    h2 = _layer(h1, src, dst, node_type2d, edge_type, eaT, ee2[:, 0], params2)
    return h2
